# baseline (speedup 1.0000x reference)
import functools

import jax
import jax.numpy as jnp
from jax import lax
from jax.experimental import pallas as pl
from jax.experimental.pallas import tpu as pltpu
from jax.experimental.pallas import tpu_sc as plsc

_D = 14336
_V = 128
_SPLIT = 80
_ROWS = 8192
_NC = 2
_NS = 16
_NW = _NC * _NS
_RPW = _ROWS // _NW
_NG = _RPW // 16
_NSA = 16
_NBB = 4


def _sc_body(table_hbm, apos_hbm, aval_hbm, bpos_hbm, bval_hbm, cnt_hbm,
             out_hbm, apos_v, aval_v, bpos_v, bval_v, cnt_v, table_sp,
             b0, b1, b2, b3, *sems):
    sid = lax.axis_index("s")
    wid = sid * _NC + lax.axis_index("c")

    asems = sems[:_NSA]
    gsems = sems[_NSA:_NSA + _NBB]
    ssems = sems[_NSA + _NBB:_NSA + 2 * _NBB]
    bufs = (b0, b1, b2, b3)

    @pl.when(sid < _SPLIT // 8)
    def _():
        pltpu.sync_copy(table_hbm.at[pl.ds(sid * 8, 8)],
                        table_sp.at[pl.ds(sid * 8, 8)])

    pltpu.sync_copy(apos_hbm.at[wid], apos_v)
    pltpu.sync_copy(aval_hbm.at[wid], aval_v)
    pltpu.sync_copy(bpos_hbm.at[wid], bpos_v)
    pltpu.sync_copy(bval_hbm.at[wid], bval_v)
    pltpu.sync_copy(cnt_hbm.at[wid], cnt_v)
    plsc.subcore_barrier()

    cvec = cnt_v[pl.ds(0, 16)]
    ga = cvec[0]
    gb = cvec[1]

    def a_group(g, first):
        pvec = apos_v[pl.ds(16 * g, 16)]
        vvec = aval_v[pl.ds(16 * g, 16)]
        for k in range(16):
            if not first:
                pltpu.make_async_copy(
                    table_sp.at[pl.ds(0, 1)],
                    out_hbm.at[pl.ds(0, 1)], asems[k]).wait()
            pltpu.make_async_copy(
                table_sp.at[pl.ds(vvec[k], 1)],
                out_hbm.at[pl.ds(pvec[k], 1)], asems[k]).start()

    def b_group(g, first):
        pvec = bpos_v[pl.ds(16 * g, 16)]
        vvec = bval_v[pl.ds(16 * g, 16)]
        for k in range(16):
            b = k % _NBB
            if not first or k >= _NBB:
                pltpu.make_async_copy(
                    bufs[b], out_hbm.at[pl.ds(0, 1)], ssems[b]).wait()
            pltpu.make_async_copy(
                table_hbm.at[pl.ds(vvec[k], 1)], bufs[b], gsems[b]).start()
            pltpu.make_async_copy(
                table_hbm.at[pl.ds(0, 1)], bufs[b], gsems[b]).wait()
            pltpu.make_async_copy(
                bufs[b], out_hbm.at[pl.ds(pvec[k], 1)], ssems[b]).start()

    @pl.when(ga > 0)
    def _():
        a_group(0, True)

    @pl.when(gb > 0)
    def _():
        b_group(0, True)

    def body(j, carry):
        g = j + 1

        @pl.when(g < ga)
        def _():
            a_group(g, False)

        @pl.when(g < gb)
        def _():
            b_group(g, False)

        return carry

    lax.fori_loop(0, _NG - 1, body, 0)

    @pl.when(ga > 0)
    def _():
        for k in range(16):
            pltpu.make_async_copy(
                table_sp.at[pl.ds(0, 1)],
                out_hbm.at[pl.ds(0, 1)], asems[k]).wait()

    @pl.when(gb > 0)
    def _():
        for b in range(_NBB):
            pltpu.make_async_copy(
                bufs[b], out_hbm.at[pl.ds(0, 1)], ssems[b]).wait()


@functools.partial(
    pl.kernel,
    mesh=plsc.VectorSubcoreMesh(core_axis_name="c", subcore_axis_name="s"),
    out_type=jax.ShapeDtypeStruct((_ROWS, _D), jnp.float32),
    scratch_types=(
        [pltpu.VMEM((_RPW,), jnp.int32)] * 4
        + [pltpu.VMEM((16,), jnp.int32),
           pltpu.VMEM_SHARED((_SPLIT, _D), jnp.float32)]
        + [pltpu.VMEM((1, _D), jnp.float32)] * _NBB
        + [pltpu.SemaphoreType.DMA] * (_NSA + 2 * _NBB)
    ),
)
def _sc_gather(table_hbm, apos, aval, bpos, bval, cnt, out_hbm, *rest):
    _sc_body(table_hbm, apos, aval, bpos, bval, cnt, out_hbm, *rest)


@jax.jit
def kernel(prefix, table):
    b, s = prefix.shape
    idx = prefix.reshape(_ROWS).astype(jnp.int32)

    vals = idx.reshape(_NW, _RPW)
    pos = jnp.arange(_ROWS, dtype=jnp.int32).reshape(_NW, _RPW)
    is_b = (vals >= _SPLIT).astype(jnp.int32)
    order = jnp.argsort(is_b, axis=1, stable=True).astype(jnp.int32)
    sv = jnp.take_along_axis(vals, order, axis=1)
    sp = jnp.take_along_axis(pos, order, axis=1)
    cnt_a = jnp.sum(1 - is_b, axis=1)
    cnt_b = _RPW - cnt_a
    i = jnp.arange(_RPW, dtype=jnp.int32)[None, :]
    a_i = jnp.clip(jnp.minimum(i, cnt_a[:, None] - 1), 0, _RPW - 1)
    b_i = jnp.clip(cnt_a[:, None] + jnp.minimum(i, cnt_b[:, None] - 1),
                   0, _RPW - 1)
    aval = jnp.take_along_axis(sv, a_i, axis=1)
    apos = jnp.take_along_axis(sp, a_i, axis=1)
    bval = jnp.take_along_axis(sv, b_i, axis=1)
    bpos = jnp.take_along_axis(sp, b_i, axis=1)
    ga = (cnt_a + 15) // 16
    gb = (cnt_b + 15) // 16
    cnt = jnp.stack([ga, gb] + [ga] * 14, axis=1).astype(jnp.int32)

    out = _sc_gather(table, apos, aval, bpos, bval, cnt)
    return out.reshape(b, s, _D)
